# baseline (speedup 1.0000x reference)
import jax
import jax.numpy as jnp
from jax.experimental import pallas as pl
from jax.experimental.pallas import tpu as pltpu

N_TOK = 16384
N_EMB = 8192
DIM = 256
TM = 256
COMMIT = 0.25


def _argmin_body(x_ref, wt_ref, idx_ref, b_ref):
    i = pl.program_id(0)

    @pl.when(i == 0)
    def _():
        wt = wt_ref[...]
        b_ref[...] = jnp.sum(wt * wt, axis=0, keepdims=True)

    x = x_ref[...]
    a = jnp.sum(x * x, axis=1, keepdims=True)
    m = jax.lax.dot_general(
        x, wt_ref[...], (((1,), (0,)), ((), ())),
        preferred_element_type=jnp.float32,
    )
    d = (a + b_ref[...]) - 2.0 * m
    rowmin = jnp.min(d, axis=1, keepdims=True)
    ids = jax.lax.broadcasted_iota(jnp.int32, d.shape, 1)
    k = d.shape[1]
    idx = jnp.min(jnp.where(d == rowmin, ids, k), axis=1)
    idx_ref[...] = idx[:, None]


def _argmin_call(x, wt):
    n, dim = x.shape
    k = wt.shape[1]
    return pl.pallas_call(
        _argmin_body,
        grid=(n // TM,),
        in_specs=[
            pl.BlockSpec((TM, dim), lambda i: (i, 0)),
            pl.BlockSpec((dim, k), lambda i: (0, 0)),
        ],
        out_specs=pl.BlockSpec((TM, 1), lambda i: (i, 0)),
        out_shape=jax.ShapeDtypeStruct((n, 1), jnp.int32),
        scratch_shapes=[pltpu.VMEM((1, k), jnp.float32)],
    )(x, wt)


def kernel(inputs, W):
    encoding_indices = _argmin_call(inputs, W.T)
    quantized = jnp.take(W, encoding_indices[:, 0], axis=0)
    q_loss = jnp.mean((quantized - inputs) ** 2)
    e_loss = jnp.mean((quantized - inputs) ** 2)
    vq_loss = q_loss + COMMIT * e_loss
    quantized_st = inputs + (quantized - inputs)
    return (quantized_st, vq_loss, encoding_indices)
